# baseline (device time: 107049 ns/iter reference)
import jax
import jax.numpy as jnp
from jax import lax
from jax.experimental import pallas as pl
from jax.experimental.pallas import tpu as pltpu

N_DEV = 8


def _ring_allgather(x2d):
    m, n = x2d.shape

    def body(x_ref, out_ref, comm_ref, send_sems, recv_sems):
        my = lax.axis_index("i")
        left = lax.rem(my + N_DEV - 1, N_DEV)
        right = lax.rem(my + 1, N_DEV)

        barrier = pltpu.get_barrier_semaphore()
        pl.semaphore_signal(barrier, inc=1, device_id=(left,),
                            device_id_type=pl.DeviceIdType.MESH)
        pl.semaphore_signal(barrier, inc=1, device_id=(right,),
                            device_id_type=pl.DeviceIdType.MESH)
        pl.semaphore_wait(barrier, 2)

        out_ref[pl.ds(my * m, m), :] = x_ref[...]
        comm_ref[0] = x_ref[...]

        for h in range(N_DEV - 1):
            s_slot = h % 2
            r_slot = (h + 1) % 2
            rdma = pltpu.make_async_remote_copy(
                src_ref=comm_ref.at[s_slot],
                dst_ref=comm_ref.at[r_slot],
                send_sem=send_sems.at[s_slot],
                recv_sem=recv_sems.at[r_slot],
                device_id=(right,),
                device_id_type=pl.DeviceIdType.MESH,
            )
            rdma.start()
            rdma.wait()
            origin = lax.rem(my + N_DEV - h - 1, N_DEV)
            out_ref[pl.ds(origin * m, m), :] = comm_ref[r_slot]

    return pl.pallas_call(
        body,
        out_shape=jax.ShapeDtypeStruct((N_DEV * m, n), x2d.dtype),
        in_specs=[pl.BlockSpec(memory_space=pltpu.VMEM)],
        out_specs=pl.BlockSpec(memory_space=pltpu.VMEM),
        scratch_shapes=[
            pltpu.VMEM((2, m, n), x2d.dtype),
            pltpu.SemaphoreType.DMA((2,)),
            pltpu.SemaphoreType.DMA((2,)),
        ],
        compiler_params=pltpu.CompilerParams(collective_id=0),
    )(x2d)


def _ring_reduce_scatter(p):
    _, m, n = p.shape

    def body(p_ref, out_ref, comm_ref, send_sems, recv_sems):
        my = lax.axis_index("i")
        left = lax.rem(my + N_DEV - 1, N_DEV)
        right = lax.rem(my + 1, N_DEV)

        barrier = pltpu.get_barrier_semaphore()
        pl.semaphore_signal(barrier, inc=1, device_id=(left,),
                            device_id_type=pl.DeviceIdType.MESH)
        pl.semaphore_signal(barrier, inc=1, device_id=(right,),
                            device_id_type=pl.DeviceIdType.MESH)
        pl.semaphore_wait(barrier, 2)

        comm_ref[0] = p_ref[lax.rem(my + N_DEV - 1, N_DEV)]

        for s in range(N_DEV - 1):
            s_slot = s % 2
            r_slot = (s + 1) % 2
            rdma = pltpu.make_async_remote_copy(
                src_ref=comm_ref.at[s_slot],
                dst_ref=comm_ref.at[r_slot],
                send_sem=send_sems.at[s_slot],
                recv_sem=recv_sems.at[r_slot],
                device_id=(right,),
                device_id_type=pl.DeviceIdType.MESH,
            )
            rdma.start()
            rdma.wait()
            c = lax.rem(my + 2 * N_DEV - 2 - s, N_DEV)
            if s < N_DEV - 2:
                comm_ref[r_slot] = comm_ref[r_slot] + p_ref[c]
            else:
                out_ref[...] = comm_ref[r_slot] + p_ref[c]

    return pl.pallas_call(
        body,
        out_shape=jax.ShapeDtypeStruct((m, n), p.dtype),
        in_specs=[pl.BlockSpec(memory_space=pltpu.VMEM)],
        out_specs=pl.BlockSpec(memory_space=pltpu.VMEM),
        scratch_shapes=[
            pltpu.VMEM((2, m, n), p.dtype),
            pltpu.SemaphoreType.DMA((2,)),
            pltpu.SemaphoreType.DMA((2,)),
        ],
        compiler_params=pltpu.CompilerParams(collective_id=1),
    )(p)


def kernel(x, Wq, K_ext, V_ext, Wo):
    i = lax.axis_index("i")
    b_loc, Sq, D = x.shape
    Dh = K_ext.shape[-1]
    Hl = Wq.shape[1] // Dh
    B = N_DEV * b_loc

    bf16 = jnp.bfloat16

    xg = _ring_allgather(x.astype(bf16).reshape(b_loc * Sq, D))

    Q = jnp.dot(xg, Wq.astype(bf16), preferred_element_type=jnp.float32)
    Q = Q.astype(bf16).reshape(B, Sq, Hl, Dh)
    K = lax.dynamic_slice_in_dim(K_ext, i * Hl, Hl, axis=2).astype(bf16)
    V = lax.dynamic_slice_in_dim(V_ext, i * Hl, Hl, axis=2).astype(bf16)
    scores = jnp.einsum('bihd,bjhd->bhij', Q, K,
                        preferred_element_type=jnp.float32) * 0.125
    w = jax.nn.softmax(scores, axis=-1)
    ctx = jnp.einsum('bhij,bjhd->bihd', w.astype(bf16), V,
                     preferred_element_type=jnp.float32)
    ctx = ctx.astype(bf16).reshape(B * Sq, Hl * Dh)
    partial = jnp.dot(ctx, Wo.astype(bf16),
                      preferred_element_type=jnp.float32)

    out = _ring_reduce_scatter(partial.reshape(N_DEV, b_loc * Sq, D))
    return out.reshape(b_loc, Sq, D)


# device time: 53190 ns/iter; 2.0126x vs baseline; 2.0126x over previous
import jax
import jax.numpy as jnp
from jax import lax
from jax.experimental import pallas as pl
from jax.experimental.pallas import tpu as pltpu

N_DEV = 8


def _allgather_weights(wq, wo):
    d, hq = wq.shape
    ho, do = wo.shape

    def body(wq_ref, wo_ref, wqf_ref, wof_ref,
             qcomm, ocomm, q_ssem, q_rsem, o_ssem, o_rsem):
        my = lax.axis_index("i")
        left = lax.rem(my + N_DEV - 1, N_DEV)
        right = lax.rem(my + 1, N_DEV)

        barrier = pltpu.get_barrier_semaphore()
        pl.semaphore_signal(barrier, inc=1, device_id=(left,),
                            device_id_type=pl.DeviceIdType.MESH)
        pl.semaphore_signal(barrier, inc=1, device_id=(right,),
                            device_id_type=pl.DeviceIdType.MESH)
        pl.semaphore_wait(barrier, 2)

        wqf_ref[:, pl.ds(my * hq, hq)] = wq_ref[...]
        wof_ref[pl.ds(my * ho, ho), :] = wo_ref[...]
        qcomm[0] = wq_ref[...]
        ocomm[0] = wo_ref[...]

        for h in range(N_DEV - 1):
            s_slot = h % 2
            r_slot = (h + 1) % 2
            q_rdma = pltpu.make_async_remote_copy(
                src_ref=qcomm.at[s_slot],
                dst_ref=qcomm.at[r_slot],
                send_sem=q_ssem.at[s_slot],
                recv_sem=q_rsem.at[r_slot],
                device_id=(right,),
                device_id_type=pl.DeviceIdType.MESH,
            )
            o_rdma = pltpu.make_async_remote_copy(
                src_ref=ocomm.at[s_slot],
                dst_ref=ocomm.at[r_slot],
                send_sem=o_ssem.at[s_slot],
                recv_sem=o_rsem.at[r_slot],
                device_id=(left,),
                device_id_type=pl.DeviceIdType.MESH,
            )
            q_rdma.start()
            o_rdma.start()
            q_rdma.wait()
            o_rdma.wait()
            q_origin = lax.rem(my + N_DEV - h - 1, N_DEV)
            o_origin = lax.rem(my + h + 1, N_DEV)
            wqf_ref[:, pl.ds(q_origin * hq, hq)] = qcomm[r_slot]
            wof_ref[pl.ds(o_origin * ho, ho), :] = ocomm[r_slot]

    return pl.pallas_call(
        body,
        out_shape=[
            jax.ShapeDtypeStruct((d, N_DEV * hq), wq.dtype),
            jax.ShapeDtypeStruct((N_DEV * ho, do), wo.dtype),
        ],
        in_specs=[
            pl.BlockSpec(memory_space=pltpu.VMEM),
            pl.BlockSpec(memory_space=pltpu.VMEM),
        ],
        out_specs=[
            pl.BlockSpec(memory_space=pltpu.VMEM),
            pl.BlockSpec(memory_space=pltpu.VMEM),
        ],
        scratch_shapes=[
            pltpu.VMEM((2, d, hq), wq.dtype),
            pltpu.VMEM((2, ho, do), wo.dtype),
            pltpu.SemaphoreType.DMA((2,)),
            pltpu.SemaphoreType.DMA((2,)),
            pltpu.SemaphoreType.DMA((2,)),
            pltpu.SemaphoreType.DMA((2,)),
        ],
        compiler_params=pltpu.CompilerParams(collective_id=0),
    )(wq, wo)


def kernel(x, Wq, K_ext, V_ext, Wo):
    i = lax.axis_index("i")
    b_loc, Sq, D = x.shape
    Hq, Dh = K_ext.shape[2:]

    bf16 = jnp.bfloat16

    WqF, WoF = _allgather_weights(Wq.astype(bf16), Wo.astype(bf16))

    Kl = lax.dynamic_slice_in_dim(K_ext, i * b_loc, b_loc, axis=0).astype(bf16)
    Vl = lax.dynamic_slice_in_dim(V_ext, i * b_loc, b_loc, axis=0).astype(bf16)
    Q = jnp.dot(x.reshape(b_loc * Sq, D).astype(bf16), WqF,
                preferred_element_type=jnp.float32)
    Q = Q.astype(bf16).reshape(b_loc, Sq, Hq, Dh)
    scores = jnp.einsum('bihd,bjhd->bhij', Q, Kl,
                        preferred_element_type=jnp.float32) * 0.125
    w = jax.nn.softmax(scores, axis=-1)
    ctx = jnp.einsum('bhij,bjhd->bihd', w.astype(bf16), Vl,
                     preferred_element_type=jnp.float32)
    ctx = ctx.astype(bf16).reshape(b_loc * Sq, Hq * Dh)
    out = jnp.dot(ctx, WoF, preferred_element_type=jnp.float32)
    return out.reshape(b_loc, Sq, D)


# device time: 50083 ns/iter; 2.1374x vs baseline; 1.0620x over previous
import jax
import jax.numpy as jnp
from jax import lax
from jax.experimental import pallas as pl
from jax.experimental.pallas import tpu as pltpu

N_DEV = 8
F32 = jnp.float32
BF16 = jnp.bfloat16


def _dot(a, b, dims):
    return lax.dot_general(a, b, (dims, ((), ())),
                           preferred_element_type=F32)


def _fused(x2d, wq, wo, K, V):
    t, d = x2d.shape
    hq = wq.shape[1]
    _, b_loc, hl, sq, dh = K.shape

    def attn_group(xv, Kv, Vv, wq_chunk, s):
        Qg = _dot(xv, wq_chunk, (((1,), (0,)))).astype(BF16)
        rows = []
        for b in range(b_loc):
            cols = []
            for h in range(hl):
                q = Qg[b * sq:(b + 1) * sq, h * dh:(h + 1) * dh]
                k = Kv[s, b, h]
                v = Vv[s, b, h]
                sc = _dot(q, k, (((1,), (1,)))) * 0.125
                m = jnp.max(sc, axis=1, keepdims=True)
                e = jnp.exp(sc - m)
                w = (e / jnp.sum(e, axis=1, keepdims=True)).astype(BF16)
                cols.append(_dot(w, v, (((1,), (0,)))).astype(BF16))
            rows.append(jnp.concatenate(cols, axis=1))
        return jnp.concatenate(rows, axis=0)

    def body(x_ref, wq_ref, wo_ref, k_ref, v_ref, out_ref,
             qbuf, obuf, ctxbuf, q_ssem, q_rsem, o_ssem, o_rsem):
        my = lax.axis_index("i")
        left = lax.rem(my + N_DEV - 1, N_DEV)
        right = lax.rem(my + 1, N_DEV)

        barrier = pltpu.get_barrier_semaphore()
        pl.semaphore_signal(barrier, inc=1, device_id=(left,),
                            device_id_type=pl.DeviceIdType.MESH)
        pl.semaphore_signal(barrier, inc=1, device_id=(right,),
                            device_id_type=pl.DeviceIdType.MESH)
        pl.semaphore_wait(barrier, 2)

        def q_rdma(h):
            return pltpu.make_async_remote_copy(
                src_ref=wq_ref if h == 0 else qbuf.at[h],
                dst_ref=qbuf.at[h + 1],
                send_sem=q_ssem.at[h],
                recv_sem=q_rsem.at[h],
                device_id=(right,),
                device_id_type=pl.DeviceIdType.MESH,
            )

        def o_rdma(h):
            return pltpu.make_async_remote_copy(
                src_ref=wo_ref if h == 0 else obuf.at[h],
                dst_ref=obuf.at[h + 1],
                send_sem=o_ssem.at[h],
                recv_sem=o_rsem.at[h],
                device_id=(left,),
                device_id_type=pl.DeviceIdType.MESH,
            )

        rdmas = {}
        rdmas[0] = (q_rdma(0), o_rdma(0))
        rdmas[0][0].start()
        rdmas[0][1].start()

        xv = x_ref[...]
        Kv = k_ref[...]
        Vv = v_ref[...]

        ctx_own = attn_group(xv, Kv, Vv, wq_ref[...], 0)
        acc = _dot(ctx_own, wo_ref[...], (((1,), (0,))))

        for h in range(N_DEV - 1):
            rq, ro = rdmas[h]
            rq.wait()
            ro.wait()
            if h < N_DEV - 2:
                rdmas[h + 1] = (q_rdma(h + 1), o_rdma(h + 1))
                rdmas[h + 1][0].start()
                rdmas[h + 1][1].start()

            ctx = attn_group(xv, Kv, Vv, qbuf[h + 1], h + 1)
            if h < 3:
                ctxbuf[h] = ctx
            else:
                acc = acc + _dot(ctx, obuf[7 - h], (((1,), (0,))))
            if h >= 4:
                acc = acc + _dot(ctxbuf[6 - h], obuf[h + 1], (((1,), (0,))))

        out_ref[...] = acc

    return pl.pallas_call(
        body,
        out_shape=jax.ShapeDtypeStruct((t, d), F32),
        in_specs=[pl.BlockSpec(memory_space=pltpu.VMEM)] * 5,
        out_specs=pl.BlockSpec(memory_space=pltpu.VMEM),
        scratch_shapes=[
            pltpu.VMEM((N_DEV, d, hq), BF16),
            pltpu.VMEM((N_DEV, hq, d), BF16),
            pltpu.VMEM((3, t, hq), BF16),
            pltpu.SemaphoreType.DMA((N_DEV - 1,)),
            pltpu.SemaphoreType.DMA((N_DEV - 1,)),
            pltpu.SemaphoreType.DMA((N_DEV - 1,)),
            pltpu.SemaphoreType.DMA((N_DEV - 1,)),
        ],
        compiler_params=pltpu.CompilerParams(collective_id=0),
    )(x2d, wq, wo, K, V)


def kernel(x, Wq, K_ext, V_ext, Wo):
    i = lax.axis_index("i")
    b_loc, Sq, D = x.shape

    Hq, Dh = K_ext.shape[2:]
    hl = Hq // N_DEV
    slots = jnp.mod(i - jnp.arange(N_DEV), N_DEV)

    def prep(A):
        Al = lax.dynamic_slice_in_dim(A, i * b_loc, b_loc, axis=0)
        Al = jnp.swapaxes(Al, 1, 2)
        Al = Al.reshape(b_loc, N_DEV, hl, Sq, Dh)
        Al = jnp.moveaxis(Al, 1, 0)
        return jnp.take(Al, slots, axis=0).astype(BF16)

    Kl = prep(K_ext)
    Vl = prep(V_ext)

    out = _fused(
        x.reshape(b_loc * Sq, D).astype(BF16),
        Wq.astype(BF16),
        Wo.astype(BF16),
        Kl, Vl,
    )
    return out.reshape(b_loc, Sq, D)


# device time: 46248 ns/iter; 2.3147x vs baseline; 1.0829x over previous
import numpy as np
import jax
import jax.numpy as jnp
from jax import lax
from jax.experimental import pallas as pl
from jax.experimental.pallas import tpu as pltpu

N_DEV = 8

PX = [1, 0, 3, 2, 5, 4, 7, 6]
PY = [3, 2, 1, 0, 7, 6, 5, 4]
PZ = [4, 5, 6, 7, 0, 1, 2, 3]


def _sigma(s, i):
    j = i
    if s & 1:
        j = PX[j]
    if s & 2:
        j = PY[j]
    if s & 4:
        j = PZ[j]
    return j


QTAB = np.array([[_sigma(s, i) for s in range(N_DEV)] for i in range(N_DEV)],
                dtype=np.int32)

F32 = jnp.float32
BF16 = jnp.bfloat16


def _dot(a, b, dims):
    return lax.dot_general(a, b, (dims, ((), ())),
                           preferred_element_type=F32)


def _fused(x2d, wq, wo, K, V, partners):
    t, d = x2d.shape
    hq = wq.shape[1]
    _, b_loc, hl, sq, dh = K.shape

    def attn_group(xv, Kv, Vv, wq_chunk, s):
        Qg = _dot(xv, wq_chunk, (((1,), (0,)))).astype(BF16)
        rows = []
        for b in range(b_loc):
            cols = []
            for h in range(hl):
                q = Qg[b * sq:(b + 1) * sq, h * dh:(h + 1) * dh]
                k = Kv[s, b, h]
                v = Vv[s, b, h]
                sc = _dot(q, k, (((1,), (1,)))) * 0.125
                m = jnp.max(sc, axis=1, keepdims=True)
                e = jnp.exp(sc - m)
                w = (e / jnp.sum(e, axis=1, keepdims=True)).astype(BF16)
                cols.append(_dot(w, v, (((1,), (0,)))).astype(BF16))
            rows.append(jnp.concatenate(cols, axis=1))
        return jnp.concatenate(rows, axis=0)

    def body(x_ref, wq_ref, wo_ref, k_ref, v_ref, prt_ref, out_ref,
             qbuf, obuf, q_ssem, q_rsem, o_ssem, o_rsem):
        px = prt_ref[0]
        py = prt_ref[1]
        pz = prt_ref[2]

        barrier = pltpu.get_barrier_semaphore()
        for p in (px, py, pz):
            pl.semaphore_signal(barrier, inc=1, device_id=(p,),
                                device_id_type=pl.DeviceIdType.MESH)
        pl.semaphore_wait(barrier, 3)

        def xfer(buf, src, dst, sem_s, sem_r, sem, target, src_ref=None):
            return pltpu.make_async_remote_copy(
                src_ref=buf.at[src] if src_ref is None else src_ref,
                dst_ref=buf.at[dst],
                send_sem=sem_s.at[sem],
                recv_sem=sem_r.at[sem],
                device_id=(target,),
                device_id_type=pl.DeviceIdType.MESH,
            )

        def q_xfer(src, dst, sem, target, src_ref=None):
            return xfer(qbuf, src, dst, q_ssem, q_rsem, sem, target, src_ref)

        def o_xfer(src, dst, sem, target, src_ref=None):
            return xfer(obuf, src, dst, o_ssem, o_rsem, sem, target, src_ref)

        r0 = [q_xfer(None, 1, 0, px, src_ref=wq_ref),
              o_xfer(None, 1, 0, py, src_ref=wo_ref)]
        for r in r0:
            r.start()

        qbuf[0] = wq_ref[...]
        obuf[0] = wo_ref[...]
        xv = x_ref[...]
        Kv = k_ref[...]
        Vv = v_ref[...]
        ctx = [None] * N_DEV
        ctx[0] = attn_group(xv, Kv, Vv, wq_ref[...], 0)
        acc = _dot(ctx[0], wo_ref[...], (((1,), (0,))))

        for r in r0:
            r.wait()

        r1q = [q_xfer(slice(0, 1), slice(2, 3), 1, py),
               q_xfer(slice(1, 2), slice(3, 4), 2, py)]
        r1o = [o_xfer(slice(0, 1), slice(2, 3), 1, pz),
               o_xfer(slice(1, 2), slice(3, 4), 2, pz)]
        for r in r1q + r1o:
            r.start()

        ctx[1] = attn_group(xv, Kv, Vv, qbuf[1], 1)

        r1q[0].wait()
        ctx[2] = attn_group(xv, Kv, Vv, qbuf[2], 2)
        acc = acc + _dot(ctx[2], obuf[1], (((1,), (0,))))
        r1q[1].wait()
        ctx[3] = attn_group(xv, Kv, Vv, qbuf[3], 3)
        for r in r1o:
            r.wait()

        r2q = [q_xfer(slice(0, 1), slice(4, 5), 3, pz),
               q_xfer(slice(1, 2), slice(5, 6), 4, pz),
               q_xfer(slice(2, 3), slice(6, 7), 5, pz),
               q_xfer(slice(3, 4), slice(7, 8), 6, pz)]
        r2o = [o_xfer(slice(0, 2), slice(4, 6), 3, px),
               o_xfer(slice(2, 4), slice(6, 8), 4, px)]
        for r in r2q + r2o:
            r.start()

        r2q[0].wait()
        ctx[4] = attn_group(xv, Kv, Vv, qbuf[4], 4)
        acc = acc + _dot(ctx[4], obuf[2], (((1,), (0,))))
        r2q[1].wait()
        ctx[5] = attn_group(xv, Kv, Vv, qbuf[5], 5)
        r2q[2].wait()
        ctx[6] = attn_group(xv, Kv, Vv, qbuf[6], 6)
        acc = acc + _dot(ctx[6], obuf[3], (((1,), (0,))))
        r2q[3].wait()
        ctx[7] = attn_group(xv, Kv, Vv, qbuf[7], 7)

        r2o[0].wait()
        acc = acc + _dot(ctx[1], obuf[4], (((1,), (0,))))
        acc = acc + _dot(ctx[3], obuf[5], (((1,), (0,))))
        r2o[1].wait()
        acc = acc + _dot(ctx[5], obuf[6], (((1,), (0,))))
        acc = acc + _dot(ctx[7], obuf[7], (((1,), (0,))))

        out_ref[...] = acc

    return pl.pallas_call(
        body,
        out_shape=jax.ShapeDtypeStruct((t, d), F32),
        in_specs=[pl.BlockSpec(memory_space=pltpu.VMEM)] * 5
        + [pl.BlockSpec(memory_space=pltpu.SMEM)],
        out_specs=pl.BlockSpec(memory_space=pltpu.VMEM),
        scratch_shapes=[
            pltpu.VMEM((N_DEV, d, hq), BF16),
            pltpu.VMEM((N_DEV, hq, d), BF16),
            pltpu.SemaphoreType.DMA((7,)),
            pltpu.SemaphoreType.DMA((7,)),
            pltpu.SemaphoreType.DMA((5,)),
            pltpu.SemaphoreType.DMA((5,)),
        ],
        compiler_params=pltpu.CompilerParams(collective_id=0),
    )(x2d, wq, wo, K, V, partners)


def kernel(x, Wq, K_ext, V_ext, Wo):
    i = lax.axis_index("i")
    b_loc, Sq, D = x.shape

    Hq, Dh = K_ext.shape[2:]
    hl = Hq // N_DEV
    slots = jnp.asarray(QTAB)[i]

    def prep(A):
        Al = lax.dynamic_slice_in_dim(A, i * b_loc, b_loc, axis=0)
        Al = Al.astype(BF16)
        Al = Al.reshape(b_loc, Sq, N_DEV, hl, Dh)
        Al = jnp.transpose(Al, (2, 0, 3, 1, 4))
        return jnp.take(Al, slots, axis=0)

    partners = jnp.stack([jnp.asarray(np.array(p, dtype=np.int32))[i]
                          for p in (PX, PY, PZ)])

    out = _fused(
        x.reshape(b_loc * Sq, D).astype(BF16),
        Wq.astype(BF16),
        Wo.astype(BF16),
        prep(K_ext), prep(V_ext),
        partners,
    )
    return out.reshape(b_loc, Sq, D)
